# baseline (speedup 1.0000x reference)
import jax
import jax.numpy as jnp
from jax.experimental import pallas as pl
from jax.experimental.pallas import tpu as pltpu

_B, _T, _H, _N = 4, 32, 8, 128
_S = 4
_L = _T // _S
_TOPK = 16
_OUT = 64
_EPS = 1e-08
_TOTAL = _S * (_N * 9 + _TOPK * 4)
_U = 2


def _softmax_phase(x_ref, u, l, pmv):
    acc = None
    for h in range(_H):
        e = jnp.exp(jnp.swapaxes(x_ref[u, l * _H + h], 0, 1))
        s = jnp.sum(e, axis=0, keepdims=True)
        p = e * (1.0 / s)
        acc = p if acc is None else acc + p
    pmv[l] = acc * (1.0 / _H)


def _stats_phase(stats_ref, u, pmv, tmpv):
    eye = (jax.lax.broadcasted_iota(jnp.int32, (_N, _N), 0)
           == jax.lax.broadcasted_iota(jnp.int32, (_N, _N), 1))

    pem = None
    for l in range(_L):
        Pl = pmv[l]
        Pc = jnp.maximum(Pl, _EPS)
        tmpv[0, l:l + 1, :] = -jnp.sum(jnp.log(Pc) * Pc, axis=0, keepdims=True)
        tmpv[1, l:l + 1, :] = jnp.sum(jnp.where(eye, Pl, 0.0), axis=0, keepdims=True)
        pem = Pl if pem is None else pem + Pl
    pem = pem * (1.0 / _L)

    d2 = None
    prev = None
    for l in range(_L):
        cur = pmv[l]
        if prev is not None:
            d = cur - prev
            tmpv[2, l - 1:l, :] = jnp.sum(jnp.abs(d), axis=0, keepdims=True)
            d2 = d * d if d2 is None else d2 + d * d
        prev = cur
    dP_mean = (pmv[_L - 1] - pmv[0]) * (1.0 / (_L - 1))
    var = (d2 - (_L - 1) * (dP_mean * dP_mean)) * (1.0 / (_L - 2))
    dP_std = jnp.sqrt(jnp.maximum(var, 0.0))
    dP_pos = jnp.maximum(dP_mean, 0.0)
    dP_neg = jnp.maximum(-dP_mean, 0.0)

    ent = tmpv[0]
    dg = tmpv[1]
    dif = tmpv[2, 0:_L - 1]
    ent_mean = jnp.mean(ent, axis=0, keepdims=True)
    ent_std = jnp.sqrt(jnp.sum((ent - ent_mean) ** 2, axis=0, keepdims=True) / (_L - 1))
    ent_range = (jnp.max(ent, axis=0, keepdims=True)
                 - jnp.min(ent, axis=0, keepdims=True))
    ent_slope = (ent[_L - 1:_L] - ent[0:1]) / (_L - 1)
    dg_mean = jnp.mean(dg, axis=0, keepdims=True)
    dg_std = jnp.sqrt(jnp.sum((dg - dg_mean) ** 2, axis=0, keepdims=True) / (_L - 1))
    dif_mean = jnp.mean(dif, axis=0, keepdims=True)
    dif_std = jnp.sqrt(jnp.sum((dif - dif_mean) ** 2, axis=0, keepdims=True) / (_L - 2))
    dif_max = jnp.max(dif, axis=0, keepdims=True)

    rows = [ent_mean, ent_std, ent_range, ent_slope,
            dif_mean, dif_std, dif_max, dg_mean, dg_std]
    for j, r in enumerate(rows):
        stats_ref[u, j:j + 1, :] = r

    score = jnp.where(eye, 0.0, jnp.abs(dP_mean))
    return score, pem, dP_std, dP_pos, dP_neg


def _topk_step(topk_ref, u, t, sc, feats, fidx, lane):
    mx = jnp.max(sc)
    fi = jnp.min(jnp.where(sc == mx, fidx, _N * _N))
    sel = fidx == fi
    v0 = jnp.sum(jnp.where(sel, feats[0], 0.0))
    v1 = jnp.sum(jnp.where(sel, feats[1], 0.0))
    v2 = jnp.sum(jnp.where(sel, feats[2], 0.0))
    v3 = jnp.sum(jnp.where(sel, feats[3], 0.0))
    rv = jnp.where(lane == 0, v0,
                   jnp.where(lane == 1, v1,
                             jnp.where(lane == 2, v2, v3)))
    rv = jnp.where(lane >= 4, 0.0, rv)
    topk_ref[u, t:t + 1, :] = rv
    return jnp.where(sel, -1.0, sc)


def _feat_kernel(x_ref, stats_ref, topk_ref, *scratch):
    pm = scratch[:_U]
    tmp = scratch[_U:]
    for l in range(_L):
        for u in range(_U):
            _softmax_phase(x_ref, u, l, pm[u])
    sc = [None] * _U
    feats = [None] * _U
    for u in range(_U):
        sc[u], *feats[u] = _stats_phase(stats_ref, u, pm[u], tmp[u])
        feats[u] = list(feats[u])
    rowi = jax.lax.broadcasted_iota(jnp.int32, (_N, _N), 0)
    coli = jax.lax.broadcasted_iota(jnp.int32, (_N, _N), 1)
    fidx = coli * _N + rowi
    lane = jax.lax.broadcasted_iota(jnp.int32, (1, _N), 1)
    for t in range(_TOPK):
        for u in range(_U):
            sc[u] = _topk_step(topk_ref, u, t, sc[u], feats[u], fidx, lane)


def _mlp_kernel(x_ref, lnw_ref, lnb_ref, w1_ref, b1_ref, w2_ref, b2_ref, o_ref):
    x = x_ref[:]
    mu = jnp.mean(x, axis=-1, keepdims=True)
    var = jnp.mean((x - mu) ** 2, axis=-1, keepdims=True)
    xn = (x - mu) * jax.lax.rsqrt(var + 1e-05) * lnw_ref[:] + lnb_ref[:]
    h = jnp.maximum(jnp.dot(xn, w1_ref[:], preferred_element_type=jnp.float32)
                    + b1_ref[:], 0.0)
    o = jnp.maximum(jnp.dot(h, w2_ref[:], preferred_element_type=jnp.float32)
                    + b2_ref[:], 0.0)
    o_ref[:] = o


def kernel(sat_scores_seq, ln_w, ln_b, W1, b1, W2, b2):
    x = sat_scores_seq.reshape(_B * _S, _L * _H, _N, _N)
    scratch = ([pltpu.VMEM((_L, _N, _N), jnp.float32) for _ in range(_U)]
               + [pltpu.VMEM((3, _L, _N), jnp.float32) for _ in range(_U)])
    stats, topk = pl.pallas_call(
        _feat_kernel,
        grid=(_B * _S // _U,),
        in_specs=[pl.BlockSpec((_U, _L * _H, _N, _N), lambda i: (i, 0, 0, 0))],
        out_specs=[pl.BlockSpec((_U, 16, _N), lambda i: (i, 0, 0)),
                   pl.BlockSpec((_U, _TOPK, _N), lambda i: (i, 0, 0))],
        out_shape=[jax.ShapeDtypeStruct((_B * _S, 16, _N), jnp.float32),
                   jax.ShapeDtypeStruct((_B * _S, _TOPK, _N), jnp.float32)],
        scratch_shapes=scratch,
    )(x)
    node = stats[:, :9, :].transpose(0, 2, 1).reshape(_B, _S, _N * 9)
    edge = topk[:, :, :4].reshape(_B, _S, _TOPK * 4)
    feats = jnp.concatenate([node, edge], axis=-1).reshape(_B, _TOTAL)
    xp = jnp.zeros((8, _TOTAL), jnp.float32).at[:_B].set(feats)
    out = pl.pallas_call(
        _mlp_kernel,
        out_shape=jax.ShapeDtypeStruct((8, _OUT), jnp.float32),
    )(xp, ln_w.reshape(1, -1), ln_b.reshape(1, -1), W1,
      b1.reshape(1, -1), W2, b2.reshape(1, -1))
    return out[:_B]


# speedup vs baseline: 1.8757x; 1.8757x over previous
import jax
import jax.numpy as jnp
from jax.experimental import pallas as pl
from jax.experimental.pallas import tpu as pltpu

_B, _T, _H, _N = 4, 32, 8, 128
_S = 4
_L = _T // _S
_TOPK = 16
_OUT = 64
_EPS = 1e-08
_TOTAL = _S * (_N * 9 + _TOPK * 4)
_U = 2


def _softmax_phase(x_ref, u, l, pmv):
    acc = None
    for h in range(_H):
        e = jnp.exp(jnp.swapaxes(x_ref[u, l * _H + h], 0, 1))
        s = jnp.sum(e, axis=0, keepdims=True)
        p = e * (1.0 / s)
        acc = p if acc is None else acc + p
    pmv[l] = acc * (1.0 / _H)


def _stats_phase(stats_ref, u, pmv, tmpv):
    eye = (jax.lax.broadcasted_iota(jnp.int32, (_N, _N), 0)
           == jax.lax.broadcasted_iota(jnp.int32, (_N, _N), 1))

    pem = None
    for l in range(_L):
        Pl = pmv[l]
        Pc = jnp.maximum(Pl, _EPS)
        tmpv[0, l:l + 1, :] = -jnp.sum(jnp.log(Pc) * Pc, axis=0, keepdims=True)
        tmpv[1, l:l + 1, :] = jnp.sum(jnp.where(eye, Pl, 0.0), axis=0, keepdims=True)
        pem = Pl if pem is None else pem + Pl
    pem = pem * (1.0 / _L)

    d2 = None
    prev = None
    for l in range(_L):
        cur = pmv[l]
        if prev is not None:
            d = cur - prev
            tmpv[2, l - 1:l, :] = jnp.sum(jnp.abs(d), axis=0, keepdims=True)
            d2 = d * d if d2 is None else d2 + d * d
        prev = cur
    dP_mean = (pmv[_L - 1] - pmv[0]) * (1.0 / (_L - 1))
    var = (d2 - (_L - 1) * (dP_mean * dP_mean)) * (1.0 / (_L - 2))
    dP_std = jnp.sqrt(jnp.maximum(var, 0.0))
    dP_pos = jnp.maximum(dP_mean, 0.0)
    dP_neg = jnp.maximum(-dP_mean, 0.0)

    ent = tmpv[0]
    dg = tmpv[1]
    dif = tmpv[2, 0:_L - 1]
    ent_mean = jnp.mean(ent, axis=0, keepdims=True)
    ent_std = jnp.sqrt(jnp.sum((ent - ent_mean) ** 2, axis=0, keepdims=True) / (_L - 1))
    ent_range = (jnp.max(ent, axis=0, keepdims=True)
                 - jnp.min(ent, axis=0, keepdims=True))
    ent_slope = (ent[_L - 1:_L] - ent[0:1]) / (_L - 1)
    dg_mean = jnp.mean(dg, axis=0, keepdims=True)
    dg_std = jnp.sqrt(jnp.sum((dg - dg_mean) ** 2, axis=0, keepdims=True) / (_L - 1))
    dif_mean = jnp.mean(dif, axis=0, keepdims=True)
    dif_std = jnp.sqrt(jnp.sum((dif - dif_mean) ** 2, axis=0, keepdims=True) / (_L - 2))
    dif_max = jnp.max(dif, axis=0, keepdims=True)

    rows = [ent_mean, ent_std, ent_range, ent_slope,
            dif_mean, dif_std, dif_max, dg_mean, dg_std]
    for j, r in enumerate(rows):
        stats_ref[u, j:j + 1, :] = r

    score = jnp.where(eye, 0.0, jnp.abs(dP_mean))
    return score, pem, dP_std, dP_pos, dP_neg


def _topk_step(topk_ref, u, t, sc, feats, fidx, lane):
    mx = jnp.max(jnp.max(sc, axis=0, keepdims=True), axis=1, keepdims=True)
    cand = jnp.where(sc == mx, fidx, _N * _N)
    fi = jnp.min(jnp.min(cand, axis=0, keepdims=True), axis=1, keepdims=True)
    sel = fidx == fi

    def _pick(f):
        m = jnp.where(sel, f, 0.0)
        return jnp.sum(jnp.sum(m, axis=0, keepdims=True), axis=1, keepdims=True)

    v0, v1, v2, v3 = _pick(feats[0]), _pick(feats[1]), _pick(feats[2]), _pick(feats[3])
    rv = jnp.where(lane == 0, v0,
                   jnp.where(lane == 1, v1,
                             jnp.where(lane == 2, v2, v3)))
    rv = jnp.where(lane >= 4, 0.0, rv)
    topk_ref[u, t:t + 1, :] = rv
    return jnp.where(sel, -1.0, sc)


def _feat_kernel(x_ref, stats_ref, topk_ref, *scratch):
    pm = scratch[:_U]
    tmp = scratch[_U:]
    for l in range(_L):
        for u in range(_U):
            _softmax_phase(x_ref, u, l, pm[u])
    sc = [None] * _U
    feats = [None] * _U
    for u in range(_U):
        sc[u], *feats[u] = _stats_phase(stats_ref, u, pm[u], tmp[u])
        feats[u] = list(feats[u])
    rowi = jax.lax.broadcasted_iota(jnp.int32, (_N, _N), 0)
    coli = jax.lax.broadcasted_iota(jnp.int32, (_N, _N), 1)
    fidx = coli * _N + rowi
    lane = jax.lax.broadcasted_iota(jnp.int32, (1, _N), 1)
    for t in range(_TOPK):
        for u in range(_U):
            sc[u] = _topk_step(topk_ref, u, t, sc[u], feats[u], fidx, lane)


def _mlp_kernel(x_ref, lnw_ref, lnb_ref, w1_ref, b1_ref, w2_ref, b2_ref, o_ref):
    x = x_ref[:]
    mu = jnp.mean(x, axis=-1, keepdims=True)
    var = jnp.mean((x - mu) ** 2, axis=-1, keepdims=True)
    xn = (x - mu) * jax.lax.rsqrt(var + 1e-05) * lnw_ref[:] + lnb_ref[:]
    h = jnp.maximum(jnp.dot(xn, w1_ref[:], preferred_element_type=jnp.float32)
                    + b1_ref[:], 0.0)
    o = jnp.maximum(jnp.dot(h, w2_ref[:], preferred_element_type=jnp.float32)
                    + b2_ref[:], 0.0)
    o_ref[:] = o


def kernel(sat_scores_seq, ln_w, ln_b, W1, b1, W2, b2):
    x = sat_scores_seq.reshape(_B * _S, _L * _H, _N, _N)
    scratch = ([pltpu.VMEM((_L, _N, _N), jnp.float32) for _ in range(_U)]
               + [pltpu.VMEM((3, _L, _N), jnp.float32) for _ in range(_U)])
    stats, topk = pl.pallas_call(
        _feat_kernel,
        grid=(_B * _S // _U,),
        in_specs=[pl.BlockSpec((_U, _L * _H, _N, _N), lambda i: (i, 0, 0, 0))],
        out_specs=[pl.BlockSpec((_U, 16, _N), lambda i: (i, 0, 0)),
                   pl.BlockSpec((_U, _TOPK, _N), lambda i: (i, 0, 0))],
        out_shape=[jax.ShapeDtypeStruct((_B * _S, 16, _N), jnp.float32),
                   jax.ShapeDtypeStruct((_B * _S, _TOPK, _N), jnp.float32)],
        scratch_shapes=scratch,
    )(x)
    node = stats[:, :9, :].transpose(0, 2, 1).reshape(_B, _S, _N * 9)
    edge = topk[:, :, :4].reshape(_B, _S, _TOPK * 4)
    feats = jnp.concatenate([node, edge], axis=-1).reshape(_B, _TOTAL)
    xp = jnp.zeros((8, _TOTAL), jnp.float32).at[:_B].set(feats)
    out = pl.pallas_call(
        _mlp_kernel,
        out_shape=jax.ShapeDtypeStruct((8, _OUT), jnp.float32),
    )(xp, ln_w.reshape(1, -1), ln_b.reshape(1, -1), W1,
      b1.reshape(1, -1), W2, b2.reshape(1, -1))
    return out[:_B]


# speedup vs baseline: 2.1413x; 1.1416x over previous
import jax
import jax.numpy as jnp
from jax.experimental import pallas as pl
from jax.experimental.pallas import tpu as pltpu

_B, _T, _H, _N = 4, 32, 8, 128
_S = 4
_L = _T // _S
_TOPK = 16
_OUT = 64
_EPS = 1e-08
_TOTAL = _S * (_N * 9 + _TOPK * 4)
_U = 4


def _softmax_phase(x_ref, u, l, pmv):
    acc = None
    for h in range(_H):
        e = jnp.exp(jnp.swapaxes(x_ref[u, l * _H + h], 0, 1))
        s = jnp.sum(e, axis=0, keepdims=True)
        p = e * (1.0 / s)
        acc = p if acc is None else acc + p
    pmv[l] = acc * (1.0 / _H)


def _stats_phase(stats_ref, u, pmv, tmpv):
    eye = (jax.lax.broadcasted_iota(jnp.int32, (_N, _N), 0)
           == jax.lax.broadcasted_iota(jnp.int32, (_N, _N), 1))

    pem = None
    for l in range(_L):
        Pl = pmv[l]
        Pc = jnp.maximum(Pl, _EPS)
        tmpv[0, l:l + 1, :] = -jnp.sum(jnp.log(Pc) * Pc, axis=0, keepdims=True)
        tmpv[1, l:l + 1, :] = jnp.sum(jnp.where(eye, Pl, 0.0), axis=0, keepdims=True)
        pem = Pl if pem is None else pem + Pl
    pem = pem * (1.0 / _L)

    d2 = None
    prev = None
    for l in range(_L):
        cur = pmv[l]
        if prev is not None:
            d = cur - prev
            tmpv[2, l - 1:l, :] = jnp.sum(jnp.abs(d), axis=0, keepdims=True)
            d2 = d * d if d2 is None else d2 + d * d
        prev = cur
    dP_mean = (pmv[_L - 1] - pmv[0]) * (1.0 / (_L - 1))
    var = (d2 - (_L - 1) * (dP_mean * dP_mean)) * (1.0 / (_L - 2))
    dP_std = jnp.sqrt(jnp.maximum(var, 0.0))
    dP_pos = jnp.maximum(dP_mean, 0.0)
    dP_neg = jnp.maximum(-dP_mean, 0.0)

    ent = tmpv[0]
    dg = tmpv[1]
    dif = tmpv[2, 0:_L - 1]
    ent_mean = jnp.mean(ent, axis=0, keepdims=True)
    ent_std = jnp.sqrt(jnp.sum((ent - ent_mean) ** 2, axis=0, keepdims=True) / (_L - 1))
    ent_range = (jnp.max(ent, axis=0, keepdims=True)
                 - jnp.min(ent, axis=0, keepdims=True))
    ent_slope = (ent[_L - 1:_L] - ent[0:1]) / (_L - 1)
    dg_mean = jnp.mean(dg, axis=0, keepdims=True)
    dg_std = jnp.sqrt(jnp.sum((dg - dg_mean) ** 2, axis=0, keepdims=True) / (_L - 1))
    dif_mean = jnp.mean(dif, axis=0, keepdims=True)
    dif_std = jnp.sqrt(jnp.sum((dif - dif_mean) ** 2, axis=0, keepdims=True) / (_L - 2))
    dif_max = jnp.max(dif, axis=0, keepdims=True)

    rows = [ent_mean, ent_std, ent_range, ent_slope,
            dif_mean, dif_std, dif_max, dg_mean, dg_std]
    for j, r in enumerate(rows):
        stats_ref[u, j:j + 1, :] = r

    score = jnp.where(eye, 0.0, jnp.abs(dP_mean))
    return score, pem, dP_std, dP_pos, dP_neg


def _topk_step(topk_ref, u, t, sc, feats, fidx, lane):
    mx = jnp.max(jnp.max(sc, axis=0, keepdims=True), axis=1, keepdims=True)
    cand = jnp.where(sc == mx, fidx, _N * _N)
    fi = jnp.min(jnp.min(cand, axis=0, keepdims=True), axis=1, keepdims=True)
    sel = fidx == fi

    def _pick(f):
        m = jnp.where(sel, f, 0.0)
        return jnp.sum(jnp.sum(m, axis=0, keepdims=True), axis=1, keepdims=True)

    v0, v1, v2, v3 = _pick(feats[0]), _pick(feats[1]), _pick(feats[2]), _pick(feats[3])
    rv = jnp.where(lane == 0, v0,
                   jnp.where(lane == 1, v1,
                             jnp.where(lane == 2, v2, v3)))
    rv = jnp.where(lane >= 4, 0.0, rv)
    topk_ref[u, t:t + 1, :] = rv
    return jnp.where(sel, -1.0, sc)


def _feat_kernel(x_ref, stats_ref, topk_ref, *scratch):
    pm = scratch[:_U]
    tmp = scratch[_U:]
    for l in range(_L):
        for u in range(_U):
            _softmax_phase(x_ref, u, l, pm[u])
    sc = [None] * _U
    feats = [None] * _U
    for u in range(_U):
        sc[u], *feats[u] = _stats_phase(stats_ref, u, pm[u], tmp[u])
        feats[u] = list(feats[u])
    rowi = jax.lax.broadcasted_iota(jnp.int32, (_N, _N), 0)
    coli = jax.lax.broadcasted_iota(jnp.int32, (_N, _N), 1)
    fidx = coli * _N + rowi
    lane = jax.lax.broadcasted_iota(jnp.int32, (1, _N), 1)
    for t in range(_TOPK):
        for u in range(_U):
            sc[u] = _topk_step(topk_ref, u, t, sc[u], feats[u], fidx, lane)


def _mlp_kernel(x_ref, lnw_ref, lnb_ref, w1_ref, b1_ref, w2_ref, b2_ref, o_ref):
    x = x_ref[:]
    mu = jnp.mean(x, axis=-1, keepdims=True)
    var = jnp.mean((x - mu) ** 2, axis=-1, keepdims=True)
    xn = (x - mu) * jax.lax.rsqrt(var + 1e-05) * lnw_ref[:] + lnb_ref[:]
    h = jnp.maximum(jnp.dot(xn, w1_ref[:], preferred_element_type=jnp.float32)
                    + b1_ref[:], 0.0)
    o = jnp.maximum(jnp.dot(h, w2_ref[:], preferred_element_type=jnp.float32)
                    + b2_ref[:], 0.0)
    o_ref[:] = o


def kernel(sat_scores_seq, ln_w, ln_b, W1, b1, W2, b2):
    x = sat_scores_seq.reshape(_B * _S, _L * _H, _N, _N)
    scratch = ([pltpu.VMEM((_L, _N, _N), jnp.float32) for _ in range(_U)]
               + [pltpu.VMEM((3, _L, _N), jnp.float32) for _ in range(_U)])
    stats, topk = pl.pallas_call(
        _feat_kernel,
        grid=(_B * _S // _U,),
        in_specs=[pl.BlockSpec((_U, _L * _H, _N, _N), lambda i: (i, 0, 0, 0))],
        out_specs=[pl.BlockSpec((_U, 16, _N), lambda i: (i, 0, 0)),
                   pl.BlockSpec((_U, _TOPK, _N), lambda i: (i, 0, 0))],
        out_shape=[jax.ShapeDtypeStruct((_B * _S, 16, _N), jnp.float32),
                   jax.ShapeDtypeStruct((_B * _S, _TOPK, _N), jnp.float32)],
        scratch_shapes=scratch,
    )(x)
    node = stats[:, :9, :].transpose(0, 2, 1).reshape(_B, _S, _N * 9)
    edge = topk[:, :, :4].reshape(_B, _S, _TOPK * 4)
    feats = jnp.concatenate([node, edge], axis=-1).reshape(_B, _TOTAL)
    xp = jnp.zeros((8, _TOTAL), jnp.float32).at[:_B].set(feats)
    out = pl.pallas_call(
        _mlp_kernel,
        out_shape=jax.ShapeDtypeStruct((8, _OUT), jnp.float32),
    )(xp, ln_w.reshape(1, -1), ln_b.reshape(1, -1), W1,
      b1.reshape(1, -1), W2, b2.reshape(1, -1))
    return out[:_B]


# speedup vs baseline: 2.1837x; 1.0198x over previous
import jax
import jax.numpy as jnp
from jax.experimental import pallas as pl
from jax.experimental.pallas import tpu as pltpu

_B, _T, _H, _N = 4, 32, 8, 128
_S = 4
_L = _T // _S
_TOPK = 16
_OUT = 64
_EPS = 1e-08
_TOTAL = _S * (_N * 9 + _TOPK * 4)
_U = 4


def _softmax_phase(x_ref, u, l, pmv):
    acc = None
    for h in range(_H):
        e = jnp.exp(jnp.swapaxes(x_ref[u, l * _H + h], 0, 1))
        s = jnp.sum(e, axis=0, keepdims=True)
        p = e * (1.0 / s)
        acc = p if acc is None else acc + p
    pmv[l] = acc * (1.0 / _H)


def _stats_phase(stats_ref, u, pmv, tmpv):
    eye = (jax.lax.broadcasted_iota(jnp.int32, (_N, _N), 0)
           == jax.lax.broadcasted_iota(jnp.int32, (_N, _N), 1))

    pem = None
    for l in range(_L):
        Pl = pmv[l]
        Pc = jnp.maximum(Pl, _EPS)
        tmpv[0, l:l + 1, :] = -jnp.sum(jnp.log(Pc) * Pc, axis=0, keepdims=True)
        tmpv[1, l:l + 1, :] = jnp.sum(jnp.where(eye, Pl, 0.0), axis=0, keepdims=True)
        pem = Pl if pem is None else pem + Pl
    pem = pem * (1.0 / _L)

    d2 = None
    prev = None
    for l in range(_L):
        cur = pmv[l]
        if prev is not None:
            d = cur - prev
            tmpv[2, l - 1:l, :] = jnp.sum(jnp.abs(d), axis=0, keepdims=True)
            d2 = d * d if d2 is None else d2 + d * d
        prev = cur
    dP_mean = (pmv[_L - 1] - pmv[0]) * (1.0 / (_L - 1))
    var = (d2 - (_L - 1) * (dP_mean * dP_mean)) * (1.0 / (_L - 2))
    dP_std = jnp.sqrt(jnp.maximum(var, 0.0))

    ent = tmpv[0]
    dg = tmpv[1]
    dif = tmpv[2, 0:_L - 1]
    ent_mean = jnp.mean(ent, axis=0, keepdims=True)
    ent_std = jnp.sqrt(jnp.sum((ent - ent_mean) ** 2, axis=0, keepdims=True) / (_L - 1))
    ent_range = (jnp.max(ent, axis=0, keepdims=True)
                 - jnp.min(ent, axis=0, keepdims=True))
    ent_slope = (ent[_L - 1:_L] - ent[0:1]) / (_L - 1)
    dg_mean = jnp.mean(dg, axis=0, keepdims=True)
    dg_std = jnp.sqrt(jnp.sum((dg - dg_mean) ** 2, axis=0, keepdims=True) / (_L - 1))
    dif_mean = jnp.mean(dif, axis=0, keepdims=True)
    dif_std = jnp.sqrt(jnp.sum((dif - dif_mean) ** 2, axis=0, keepdims=True) / (_L - 2))
    dif_max = jnp.max(dif, axis=0, keepdims=True)

    rows = [ent_mean, ent_std, ent_range, ent_slope,
            dif_mean, dif_std, dif_max, dg_mean, dg_std]
    for j, r in enumerate(rows):
        stats_ref[u, j:j + 1, :] = r

    score = jnp.where(eye, 0.0, jnp.abs(dP_mean))
    return score, pem, dP_std, dP_mean


def _topk_step(topk_ref, u, t, sc, feats, fidx, lane):
    mx = jnp.max(jnp.max(sc, axis=0, keepdims=True), axis=1, keepdims=True)
    cand = jnp.where(sc == mx, fidx, _N * _N)
    fi = jnp.min(jnp.min(cand, axis=0, keepdims=True), axis=1, keepdims=True)
    sel = fidx == fi

    def _pick(f):
        m = jnp.where(sel, f, 0.0)
        return jnp.sum(jnp.sum(m, axis=0, keepdims=True), axis=1, keepdims=True)

    v0, v1 = _pick(feats[0]), _pick(feats[1])
    dm = _pick(feats[2])
    v2 = jnp.maximum(dm, 0.0)
    v3 = jnp.maximum(-dm, 0.0)
    rv = jnp.where(lane == 0, v0,
                   jnp.where(lane == 1, v1,
                             jnp.where(lane == 2, v2, v3)))
    rv = jnp.where(lane >= 4, 0.0, rv)
    topk_ref[u, t:t + 1, :] = rv
    return jnp.where(sel, -1.0, sc)


def _feat_kernel(x_ref, stats_ref, topk_ref, *scratch):
    pm = scratch[:_U]
    tmp = scratch[_U:]
    for l in range(_L):
        for u in range(_U):
            _softmax_phase(x_ref, u, l, pm[u])
    sc = [None] * _U
    feats = [None] * _U
    for u in range(_U):
        sc[u], *feats[u] = _stats_phase(stats_ref, u, pm[u], tmp[u])
        feats[u] = list(feats[u])
    rowi = jax.lax.broadcasted_iota(jnp.int32, (_N, _N), 0)
    coli = jax.lax.broadcasted_iota(jnp.int32, (_N, _N), 1)
    fidx = coli * _N + rowi
    lane = jax.lax.broadcasted_iota(jnp.int32, (1, _N), 1)
    for t in range(_TOPK):
        for u in range(_U):
            sc[u] = _topk_step(topk_ref, u, t, sc[u], feats[u], fidx, lane)


def _mlp_kernel(x_ref, lnw_ref, lnb_ref, w1_ref, b1_ref, w2_ref, b2_ref, o_ref):
    x = x_ref[:]
    mu = jnp.mean(x, axis=-1, keepdims=True)
    var = jnp.mean((x - mu) ** 2, axis=-1, keepdims=True)
    xn = (x - mu) * jax.lax.rsqrt(var + 1e-05) * lnw_ref[:] + lnb_ref[:]
    h = jnp.maximum(jnp.dot(xn, w1_ref[:], preferred_element_type=jnp.float32)
                    + b1_ref[:], 0.0)
    o = jnp.maximum(jnp.dot(h, w2_ref[:], preferred_element_type=jnp.float32)
                    + b2_ref[:], 0.0)
    o_ref[:] = o


def kernel(sat_scores_seq, ln_w, ln_b, W1, b1, W2, b2):
    x = sat_scores_seq.reshape(_B * _S, _L * _H, _N, _N)
    scratch = ([pltpu.VMEM((_L, _N, _N), jnp.float32) for _ in range(_U)]
               + [pltpu.VMEM((3, _L, _N), jnp.float32) for _ in range(_U)])
    stats, topk = pl.pallas_call(
        _feat_kernel,
        grid=(_B * _S // _U,),
        in_specs=[pl.BlockSpec((_U, _L * _H, _N, _N), lambda i: (i, 0, 0, 0))],
        out_specs=[pl.BlockSpec((_U, 16, _N), lambda i: (i, 0, 0)),
                   pl.BlockSpec((_U, _TOPK, _N), lambda i: (i, 0, 0))],
        out_shape=[jax.ShapeDtypeStruct((_B * _S, 16, _N), jnp.float32),
                   jax.ShapeDtypeStruct((_B * _S, _TOPK, _N), jnp.float32)],
        scratch_shapes=scratch,
    )(x)
    node = stats[:, :9, :].transpose(0, 2, 1).reshape(_B, _S, _N * 9)
    edge = topk[:, :, :4].reshape(_B, _S, _TOPK * 4)
    feats = jnp.concatenate([node, edge], axis=-1).reshape(_B, _TOTAL)
    xp = jnp.zeros((8, _TOTAL), jnp.float32).at[:_B].set(feats)
    out = pl.pallas_call(
        _mlp_kernel,
        out_shape=jax.ShapeDtypeStruct((8, _OUT), jnp.float32),
    )(xp, ln_w.reshape(1, -1), ln_b.reshape(1, -1), W1,
      b1.reshape(1, -1), W2, b2.reshape(1, -1))
    return out[:_B]


# speedup vs baseline: 2.2376x; 1.0247x over previous
import jax
import jax.numpy as jnp
from jax.experimental import pallas as pl
from jax.experimental.pallas import tpu as pltpu

_B, _T, _H, _N = 4, 32, 8, 128
_S = 4
_L = _T // _S
_TOPK = 16
_OUT = 64
_EPS = 1e-08
_TOTAL = _S * (_N * 9 + _TOPK * 4)
_U = 4


def _softmax_phase(x_ref, u, l, pmv):
    acc = None
    for h in range(_H):
        e = jnp.exp(x_ref[u, l * _H + h])
        s = jnp.sum(e, axis=-1, keepdims=True)
        p = e / s
        acc = p if acc is None else acc + p
    pmv[l] = jnp.swapaxes(acc, 0, 1) * (1.0 / _H)


def _stats_phase(stats_ref, u, pmv, tmpv):
    eye = (jax.lax.broadcasted_iota(jnp.int32, (_N, _N), 0)
           == jax.lax.broadcasted_iota(jnp.int32, (_N, _N), 1))

    pem = None
    for l in range(_L):
        Pl = pmv[l]
        Pc = jnp.maximum(Pl, _EPS)
        tmpv[0, l:l + 1, :] = -jnp.sum(jnp.log(Pc) * Pc, axis=0, keepdims=True)
        tmpv[1, l:l + 1, :] = jnp.sum(jnp.where(eye, Pl, 0.0), axis=0, keepdims=True)
        pem = Pl if pem is None else pem + Pl
    pem = pem * (1.0 / _L)

    d2 = None
    prev = None
    for l in range(_L):
        cur = pmv[l]
        if prev is not None:
            d = cur - prev
            tmpv[2, l - 1:l, :] = jnp.sum(jnp.abs(d), axis=0, keepdims=True)
            d2 = d * d if d2 is None else d2 + d * d
        prev = cur
    dP_mean = (pmv[_L - 1] - pmv[0]) * (1.0 / (_L - 1))
    var = (d2 - (_L - 1) * (dP_mean * dP_mean)) * (1.0 / (_L - 2))
    dP_std = jnp.sqrt(jnp.maximum(var, 0.0))

    ent = tmpv[0]
    dg = tmpv[1]
    dif = tmpv[2, 0:_L - 1]
    ent_mean = jnp.mean(ent, axis=0, keepdims=True)
    ent_std = jnp.sqrt(jnp.sum((ent - ent_mean) ** 2, axis=0, keepdims=True) / (_L - 1))
    ent_range = (jnp.max(ent, axis=0, keepdims=True)
                 - jnp.min(ent, axis=0, keepdims=True))
    ent_slope = (ent[_L - 1:_L] - ent[0:1]) / (_L - 1)
    dg_mean = jnp.mean(dg, axis=0, keepdims=True)
    dg_std = jnp.sqrt(jnp.sum((dg - dg_mean) ** 2, axis=0, keepdims=True) / (_L - 1))
    dif_mean = jnp.mean(dif, axis=0, keepdims=True)
    dif_std = jnp.sqrt(jnp.sum((dif - dif_mean) ** 2, axis=0, keepdims=True) / (_L - 2))
    dif_max = jnp.max(dif, axis=0, keepdims=True)

    rows = [ent_mean, ent_std, ent_range, ent_slope,
            dif_mean, dif_std, dif_max, dg_mean, dg_std]
    for j, r in enumerate(rows):
        stats_ref[u, j:j + 1, :] = r

    score = jnp.where(eye, 0.0, jnp.abs(dP_mean))
    return score, pem, dP_std, dP_mean


def _topk_step(topk_ref, u, t, sc, feats, fidx, lane):
    mx = jnp.max(jnp.max(sc, axis=0, keepdims=True), axis=1, keepdims=True)
    cand = jnp.where(sc == mx, fidx, _N * _N)
    fi = jnp.min(jnp.min(cand, axis=0, keepdims=True), axis=1, keepdims=True)
    sel = fidx == fi

    def _pick(f):
        m = jnp.where(sel, f, 0.0)
        return jnp.sum(jnp.sum(m, axis=0, keepdims=True), axis=1, keepdims=True)

    v0, v1 = _pick(feats[0]), _pick(feats[1])
    dm = _pick(feats[2])
    v2 = jnp.maximum(dm, 0.0)
    v3 = jnp.maximum(-dm, 0.0)
    rv = jnp.where(lane == 0, v0,
                   jnp.where(lane == 1, v1,
                             jnp.where(lane == 2, v2, v3)))
    rv = jnp.where(lane >= 4, 0.0, rv)
    topk_ref[u, t:t + 1, :] = rv
    return jnp.where(sel, -1.0, sc)


def _feat_kernel(x_ref, stats_ref, topk_ref, *scratch):
    pm = scratch[:_U]
    tmp = scratch[_U:]
    for l in range(_L):
        for u in range(_U):
            _softmax_phase(x_ref, u, l, pm[u])
    sc = [None] * _U
    feats = [None] * _U
    for u in range(_U):
        sc[u], *feats[u] = _stats_phase(stats_ref, u, pm[u], tmp[u])
        feats[u] = list(feats[u])
    rowi = jax.lax.broadcasted_iota(jnp.int32, (_N, _N), 0)
    coli = jax.lax.broadcasted_iota(jnp.int32, (_N, _N), 1)
    fidx = coli * _N + rowi
    lane = jax.lax.broadcasted_iota(jnp.int32, (1, _N), 1)
    for t in range(_TOPK):
        for u in range(_U):
            sc[u] = _topk_step(topk_ref, u, t, sc[u], feats[u], fidx, lane)


def _mlp_kernel(x_ref, lnw_ref, lnb_ref, w1_ref, b1_ref, w2_ref, b2_ref, o_ref):
    x = x_ref[:]
    mu = jnp.mean(x, axis=-1, keepdims=True)
    var = jnp.mean((x - mu) ** 2, axis=-1, keepdims=True)
    xn = (x - mu) * jax.lax.rsqrt(var + 1e-05) * lnw_ref[:] + lnb_ref[:]
    h = jnp.maximum(jnp.dot(xn, w1_ref[:], preferred_element_type=jnp.float32)
                    + b1_ref[:], 0.0)
    o = jnp.maximum(jnp.dot(h, w2_ref[:], preferred_element_type=jnp.float32)
                    + b2_ref[:], 0.0)
    o_ref[:] = o


def kernel(sat_scores_seq, ln_w, ln_b, W1, b1, W2, b2):
    x = sat_scores_seq.reshape(_B * _S, _L * _H, _N, _N)
    scratch = ([pltpu.VMEM((_L, _N, _N), jnp.float32) for _ in range(_U)]
               + [pltpu.VMEM((3, _L, _N), jnp.float32) for _ in range(_U)])
    stats, topk = pl.pallas_call(
        _feat_kernel,
        grid=(_B * _S // _U,),
        in_specs=[pl.BlockSpec((_U, _L * _H, _N, _N), lambda i: (i, 0, 0, 0))],
        out_specs=[pl.BlockSpec((_U, 16, _N), lambda i: (i, 0, 0)),
                   pl.BlockSpec((_U, _TOPK, _N), lambda i: (i, 0, 0))],
        out_shape=[jax.ShapeDtypeStruct((_B * _S, 16, _N), jnp.float32),
                   jax.ShapeDtypeStruct((_B * _S, _TOPK, _N), jnp.float32)],
        scratch_shapes=scratch,
    )(x)
    node = stats[:, :9, :].transpose(0, 2, 1).reshape(_B, _S, _N * 9)
    edge = topk[:, :, :4].reshape(_B, _S, _TOPK * 4)
    feats = jnp.concatenate([node, edge], axis=-1).reshape(_B, _TOTAL)
    xp = jnp.zeros((8, _TOTAL), jnp.float32).at[:_B].set(feats)
    out = pl.pallas_call(
        _mlp_kernel,
        out_shape=jax.ShapeDtypeStruct((8, _OUT), jnp.float32),
    )(xp, ln_w.reshape(1, -1), ln_b.reshape(1, -1), W1,
      b1.reshape(1, -1), W2, b2.reshape(1, -1))
    return out[:_B]
